# Initial kernel scaffold; baseline (speedup 1.0000x reference)
import jax
import jax.numpy as jnp
from jax.experimental import pallas as pl


def kernel(x, edge_index, W1, b1, W2, b2):
    raise NotImplementedError("write your pallas kernel here")


# speedup vs baseline: 11.5807x; 11.5807x over previous
import jax
import jax.numpy as jnp
from jax import lax
from jax.experimental import pallas as pl
from jax.experimental.pallas import tpu as pltpu
from jax.experimental.pallas import tpu_sc as plsc

N = 10000
E = 320000
NT = 32
CH = 80
K = 128
EP = NT * CH * K
NACC = 10240
STRIPE = 640
SUB = 32
D1 = 128
D2 = 48
RB = 1000

_MESH = plsc.VectorSubcoreMesh(core_axis_name="c", subcore_axis_name="s")


def _deg_body(col3, zeros_hbm, ones_hbm, out, cbuf, ones_v, zbuf, acc):
    c = lax.axis_index("c")
    s = lax.axis_index("s")
    wid = c * 16 + s
    pltpu.sync_copy(zeros_hbm, zbuf)
    pltpu.sync_copy(ones_hbm, ones_v)
    pltpu.sync_copy(zbuf, acc.at[pl.ds(s * STRIPE, STRIPE)])
    plsc.subcore_barrier()
    pltpu.sync_copy(col3.at[wid], cbuf)

    def step(ch, carry):
        pltpu.sync_copy(ones_v, acc.at[cbuf.at[ch]], add=True)
        return carry

    lax.fori_loop(0, CH, step, 0)
    plsc.subcore_barrier()
    pltpu.sync_copy(acc.at[pl.ds(s * STRIPE, STRIPE)], zbuf)
    pltpu.sync_copy(zbuf, out.at[pl.ds(c * NACC + s * STRIPE, STRIPE)])


def _deg_call(col3, zeros16, ones16):
    return pl.kernel(
        _deg_body,
        out_type=jax.ShapeDtypeStruct((2 * NACC, 16), jnp.float32),
        mesh=_MESH,
        scratch_types=[
            pltpu.VMEM((CH, K), jnp.int32),
            pltpu.VMEM((K, 16), jnp.float32),
            pltpu.VMEM((STRIPE, 16), jnp.float32),
            pltpu.VMEM_SHARED((NACC, 16), jnp.float32),
        ],
        compiler_params=pltpu.CompilerParams(use_tc_tiling_on_sc=False),
    )(col3, zeros16, ones16)


def _prop_body(eb, z, zeros_hbm, out,
               ibuf, g0, g1, zbuf, acc, gs0, gs1, ss0, ss1):
    c = lax.axis_index("c")
    s = lax.axis_index("s")
    wid = c * 16 + s
    pltpu.sync_copy(zeros_hbm, zbuf)
    for kk in range(STRIPE // SUB):
        pltpu.sync_copy(zbuf, acc.at[pl.ds(s * STRIPE + kk * SUB, SUB)])
    plsc.subcore_barrier()

    gb = (g0, g1)
    gs = (gs0, gs1)
    ss = (ss0, ss1)
    pltpu.sync_copy(eb.at[wid, 0], ibuf.at[0])
    pltpu.async_copy(z.at[ibuf.at[0, 0]], g0, gs0)

    def pair(i, carry):
        for b in range(2):
            ch = 2 * i + b
            pltpu.make_async_copy(z.at[ibuf.at[b, 0]], gb[b], gs[b]).wait()
            pltpu.async_copy(gb[b], acc.at[ibuf.at[b, 1]], ss[b], add=True)

            @pl.when(ch >= 1)
            def _():
                pltpu.make_async_copy(gb[1 - b], acc.at[ibuf.at[1 - b, 1]],
                                      ss[1 - b]).wait()

            @pl.when(ch + 1 < CH)
            def _():
                pltpu.sync_copy(eb.at[wid, ch + 1], ibuf.at[1 - b])
                pltpu.async_copy(z.at[ibuf.at[1 - b, 0]], gb[1 - b], gs[1 - b])

            return_carry = carry
        return return_carry

    lax.fori_loop(0, CH // 2, pair, 0)
    pltpu.make_async_copy(gb[1], acc.at[ibuf.at[1, 1]], ss[1]).wait()
    plsc.subcore_barrier()
    for kk in range(STRIPE // SUB):
        off = s * STRIPE + kk * SUB
        pltpu.sync_copy(acc.at[pl.ds(off, SUB)], zbuf)
        pltpu.sync_copy(zbuf, out.at[pl.ds(c * NACC + off, SUB)])


def _prop_call(eb, z, zeros_d, d):
    return pl.kernel(
        _prop_body,
        out_type=jax.ShapeDtypeStruct((2 * NACC, d), jnp.float32),
        mesh=_MESH,
        scratch_types=[
            pltpu.VMEM((2, 2, K), jnp.int32),
            pltpu.VMEM((K, d), jnp.float32),
            pltpu.VMEM((K, d), jnp.float32),
            pltpu.VMEM((SUB, d), jnp.float32),
            pltpu.VMEM_SHARED((NACC, d), jnp.float32),
            pltpu.SemaphoreType.DMA,
            pltpu.SemaphoreType.DMA,
            pltpu.SemaphoreType.DMA,
            pltpu.SemaphoreType.DMA,
        ],
        compiler_params=pltpu.CompilerParams(use_tc_tiling_on_sc=False),
    )(eb, z, zeros_d)


def _tc_a_body(d0, d1, x, z1):
    dis = lax.rsqrt(d0[...] + d1[...] + 1.0)
    z1[...] = x[...] * dis


def _tc_b_body(d0, d1, s0, s1, z1v, w1, b1, w2, z2):
    dis = lax.rsqrt(d0[...] + d1[...] + 1.0)
    u = dis * (s0[...] + s1[...] + z1v[...])
    h = jnp.maximum(jnp.dot(u, w1[...], preferred_element_type=jnp.float32)
                    + b1[...], 0.0)
    y2 = jnp.dot(h, w2[...], preferred_element_type=jnp.float32)
    z2[...] = dis * y2


def _tc_c_body(d0, d1, t0, t1, z2v, b2, o):
    dis = lax.rsqrt(d0[...] + d1[...] + 1.0)
    v = jnp.maximum(dis * (t0[...] + t1[...] + z2v[...]) + b2[...], 0.0)
    mask = lax.broadcasted_iota(jnp.int32, (RB, D2), 1) < 40
    vm = jnp.where(mask, v, -1e30)
    m = jnp.max(vm, axis=-1, keepdims=True)
    lse = jnp.log(jnp.sum(jnp.exp(vm - m), axis=-1, keepdims=True))
    o[...] = vm - m - lse


def _col_spec(w):
    return pl.BlockSpec((RB, w), lambda i: (i, 0))


def _full_spec(r, w):
    return pl.BlockSpec((r, w), lambda i: (0, 0))


def _tc_a(d0, d1, x):
    return pl.pallas_call(
        _tc_a_body,
        grid=(N // RB,),
        in_specs=[_col_spec(1), _col_spec(1), _col_spec(D1)],
        out_specs=_col_spec(D1),
        out_shape=jax.ShapeDtypeStruct((N, D1), jnp.float32),
    )(d0, d1, x)


def _tc_b(d0, d1, s0, s1, z1, w1, b1, w2):
    return pl.pallas_call(
        _tc_b_body,
        grid=(N // RB,),
        in_specs=[_col_spec(1), _col_spec(1), _col_spec(D1), _col_spec(D1),
                  _col_spec(D1), _full_spec(D1, D1), _full_spec(1, D1),
                  _full_spec(D1, D2)],
        out_specs=_col_spec(D2),
        out_shape=jax.ShapeDtypeStruct((N, D2), jnp.float32),
    )(d0, d1, s0, s1, z1, w1, b1, w2)


def _tc_c(d0, d1, t0, t1, z2, b2):
    return pl.pallas_call(
        _tc_c_body,
        grid=(N // RB,),
        in_specs=[_col_spec(1), _col_spec(1), _col_spec(D2), _col_spec(D2),
                  _col_spec(D2), _full_spec(1, D2)],
        out_specs=_col_spec(D2),
        out_shape=jax.ShapeDtypeStruct((N, D2), jnp.float32),
    )(d0, d1, t0, t1, z2, b2)


def kernel(x, edge_index, W1, b1, W2, b2):
    ei = edge_index.astype(jnp.int32)
    pad_r = jnp.zeros((EP - E,), jnp.int32)
    pad_c = jnp.full((EP - E,), N, jnp.int32)
    row3 = jnp.concatenate([ei[0], pad_r]).reshape(NT, CH, K)
    col3 = jnp.concatenate([ei[1], pad_c]).reshape(NT, CH, K)
    eb = jnp.stack([row3, col3], axis=2)

    zeros16 = jnp.zeros((STRIPE, 16), jnp.float32)
    ones16 = jnp.ones((K, 16), jnp.float32)
    deg_out = _deg_call(col3, zeros16, ones16)
    d0 = deg_out[0:N, 0:1]
    d1 = deg_out[NACC:NACC + N, 0:1]

    z1 = _tc_a(d0, d1, x)
    s_out = _prop_call(eb, z1, jnp.zeros((SUB, D1), jnp.float32), D1)
    s0 = s_out[0:N]
    s1 = s_out[NACC:NACC + N]

    w2p = jnp.pad(W2, ((0, 0), (0, D2 - 40)))
    b2p = jnp.pad(b2, (0, D2 - 40)).reshape(1, D2)
    z2 = _tc_b(d0, d1, s0, s1, z1, W1, b1.reshape(1, D1), w2p)

    t_out = _prop_call(eb, z2, jnp.zeros((SUB, D2), jnp.float32), D2)
    t0 = t_out[0:N]
    t1 = t_out[NACC:NACC + N]

    o = _tc_c(d0, d1, t0, t1, z2, b2p)
    return o[:, :40]
